# Initial kernel scaffold; baseline (speedup 1.0000x reference)
import jax
import jax.numpy as jnp
from jax.experimental import pallas as pl


def kernel(triples, edge_index, entity_emb, W1, a_src1, a_dst1, b1, W2, a_src2, a_dst2, b2, res_W, res_b, ln_g, ln_b, rel_emb):
    raise NotImplementedError("write your pallas kernel here")


# speedup vs baseline: 34.7701x; 34.7701x over previous
import functools

import jax
import jax.numpy as jnp
from jax import lax
from jax.experimental import pallas as pl
from jax.experimental.pallas import tpu as pltpu
from jax.experimental.pallas import tpu_sc as plsc

N = 10000
NREL = 4
E = 80000
EMB = 128
HID = 64
HEADS = 2
OUT = 256
B = 16384

NCHUNK = E // 128
MAXCH = (NCHUNK + 15) // 16
TN = 1000
DEN_PAD = 10240

_i32 = jnp.int32
_f32 = jnp.float32


def _edge_body(Fh, src_hbm, dst_hbm, h0_hbm, h1_hbm, as_hbm, ad_hbm,
               acc_hbm, den_hbm,
               asl, adl, srcb, dstb, ech, gbuf, zbuf, den_sh, acc_sh, gsem):
    c = lax.axis_index("c")
    s = lax.axis_index("s")
    base = c * N
    zero16 = jnp.zeros((16,), _f32)

    pltpu.sync_copy(as_hbm.at[pl.ds(base, N)], asl)
    pltpu.sync_copy(ad_hbm.at[pl.ds(base, N)], adl)

    def _zg(i, carry):
        for f in range(Fh // 16):
            gbuf[i, pl.ds(f * 16, 16)] = zero16
        return carry
    lax.fori_loop(0, 128, _zg, 0)
    def _zz(i, carry):
        zbuf[pl.ds(i * 16, 16)] = zero16
        return carry
    lax.fori_loop(0, 40, _zz, 0)

    r0 = s * 625
    for kk in range(4):
        pltpu.sync_copy(gbuf, acc_sh.at[pl.ds(r0 + kk * 128, 128)])
    pltpu.sync_copy(gbuf.at[pl.ds(0, 113)], acc_sh.at[pl.ds(r0 + 512, 113)])
    pltpu.sync_copy(zbuf, den_sh.at[pl.ds(s * 640, 640)])

    plsc.subcore_barrier()

    def _chunk(j, carry):
        ci = s + 16 * j

        @pl.when(ci < NCHUNK)
        def _():
            pltpu.sync_copy(src_hbm.at[pl.ds(ci * 128, 128)], srcb)
            pltpu.sync_copy(dst_hbm.at[pl.ds(ci * 128, 128)], dstb)

            @pl.when(c == 0)
            def _():
                pltpu.async_copy(h0_hbm.at[srcb], gbuf, gsem)

            @pl.when(c == 1)
            def _():
                pltpu.async_copy(h1_hbm.at[srcb], gbuf, gsem)

            for g in range(8):
                sv = srcb[pl.ds(g * 16, 16)]
                dv = dstb[pl.ds(g * 16, 16)]
                av = plsc.load_gather(asl, [sv]) + plsc.load_gather(adl, [dv])
                av = jnp.where(av >= 0.0, av, av * jnp.float32(0.2))
                ech[pl.ds(g * 16, 16)] = jnp.exp(av)

            pltpu.sync_copy(ech, den_sh.at[dstb], add=True)

            pltpu.make_async_copy(h0_hbm.at[srcb], gbuf, gsem).wait()

            def _scale(rw, carry2):
                ev = plsc.load_gather(ech, [jnp.full((16,), rw, _i32)])
                for f in range(Fh // 16):
                    gbuf[rw, pl.ds(f * 16, 16)] = (
                        gbuf[rw, pl.ds(f * 16, 16)] * ev)
                return carry2
            lax.fori_loop(0, 128, _scale, 0)

            pltpu.sync_copy(gbuf, acc_sh.at[dstb], add=True)
        return carry
    lax.fori_loop(0, MAXCH, _chunk, 0)

    plsc.subcore_barrier()

    pltpu.sync_copy(acc_sh.at[pl.ds(r0, 625)],
                    acc_hbm.at[pl.ds(base + r0, 625)])

    @pl.when(s == 0)
    def _():
        pltpu.sync_copy(den_sh.at[pl.ds(0, N)], den_hbm.at[c])


@functools.cache
def _make_edge_kernel(Fh):
    mesh = plsc.VectorSubcoreMesh(core_axis_name="c", subcore_axis_name="s")
    return pl.kernel(
        functools.partial(_edge_body, Fh),
        out_type=[
            jax.ShapeDtypeStruct((2 * N, Fh), _f32),
            jax.ShapeDtypeStruct((2, N), _f32),
        ],
        mesh=mesh,
        scratch_types=[
            pltpu.VMEM((N,), _f32),
            pltpu.VMEM((N,), _f32),
            pltpu.VMEM((128,), _i32),
            pltpu.VMEM((128,), _i32),
            pltpu.VMEM((128,), _f32),
            pltpu.VMEM((128, Fh), _f32),
            pltpu.VMEM((640,), _f32),
            pltpu.VMEM_SHARED((DEN_PAD,), _f32),
            pltpu.VMEM_SHARED((N, Fh), _f32),
            pltpu.SemaphoreType.DMA,
        ],
        compiler_params=pltpu.CompilerParams(use_tc_tiling_on_sc=False, needs_layout_passes=False),
    )


def _score_body(hcol_hbm, tcol_hbm, wrow_hbm, ent_hbm, out_hbm,
                hl, tl, ehb, etb, wrb, pb, sb, sem1, sem2):
    c = lax.axis_index("c")
    s = lax.axis_index("s")
    w = c * 16 + s
    off = w * 512

    pltpu.sync_copy(hcol_hbm.at[pl.ds(off, 512)], hl)
    pltpu.sync_copy(tcol_hbm.at[pl.ds(off, 512)], tl)

    def _chunk(q, carry):
        cp1 = pltpu.async_copy(ent_hbm.at[hl.at[pl.ds(q * 64, 64)]], ehb, sem1)
        cp2 = pltpu.async_copy(ent_hbm.at[tl.at[pl.ds(q * 64, 64)]], etb, sem2)
        pltpu.sync_copy(wrow_hbm.at[pl.ds(off + q * 64, 64)], wrb)
        cp1.wait()
        cp2.wait()

        def _triple(j, carry2):
            acc = jnp.zeros((16,), _f32)
            for f in range(OUT // 16):
                acc = acc + (ehb[j, pl.ds(f * 16, 16)]
                             * etb[j, pl.ds(f * 16, 16)]
                             * wrb[j, pl.ds(f * 16, 16)])
            pb[q * 64 + j] = acc
            return carry2
        lax.fori_loop(0, 64, _triple, 0)
        return carry
    lax.fori_loop(0, 8, _chunk, 0)

    iota16 = lax.iota(_i32, 16)

    def _red(jg, carry):
        rows = iota16 + jg * 16
        tot = jnp.zeros((16,), _f32)
        for f in range(16):
            tot = tot + plsc.load_gather(pb, [rows, jnp.full((16,), f, _i32)])
        sb[pl.ds(jg * 16, 16)] = tot
        return carry
    lax.fori_loop(0, 32, _red, 0)

    pltpu.sync_copy(sb, out_hbm.at[pl.ds(off, 512)])


@functools.cache
def _make_score_kernel():
    mesh = plsc.VectorSubcoreMesh(core_axis_name="c", subcore_axis_name="s")
    return pl.kernel(
        _score_body,
        out_type=jax.ShapeDtypeStruct((B,), _f32),
        mesh=mesh,
        scratch_types=[
            pltpu.VMEM((512,), _i32),
            pltpu.VMEM((512,), _i32),
            pltpu.VMEM((64, OUT), _f32),
            pltpu.VMEM((64, OUT), _f32),
            pltpu.VMEM((64, OUT), _f32),
            pltpu.VMEM((512, 16), _f32),
            pltpu.VMEM((512,), _f32),
            pltpu.SemaphoreType.DMA,
            pltpu.SemaphoreType.DMA,
        ],
        compiler_params=pltpu.CompilerParams(use_tc_tiling_on_sc=False, needs_layout_passes=False),
    )


def _wsel_body(rc_ref, emb_ref, o_ref):
    rc = rc_ref[...]
    out = jnp.zeros((rc.shape[0], OUT), _f32)
    for r in range(NREL):
        out = out + jnp.where(rc == r, 1.0, 0.0) * emb_ref[r][None, :]
    o_ref[...] = out


def _run_wsel(rcol2d, rel_emb):
    TB = 2048
    return pl.pallas_call(
        _wsel_body,
        grid=(B // TB,),
        in_specs=[
            pl.BlockSpec((TB, 1), lambda i: (i, 0)),
            pl.BlockSpec((NREL, OUT), lambda i: (0, 0)),
        ],
        out_specs=pl.BlockSpec((TB, OUT), lambda i: (i, 0)),
        out_shape=jax.ShapeDtypeStruct((B, OUT), _f32),
    )(rcol2d, rel_emb)


def _k1_body(x_ref, w_ref, as_ref, ad_ref, h0_ref, h1_ref, a_ref):
    xb = x_ref[...]
    h = jnp.dot(xb, w_ref[0], preferred_element_type=_f32)
    h0 = h[:, :HID]
    h1 = h[:, HID:]
    h0_ref[0] = h0
    h1_ref[0] = h1
    a_s = as_ref[0, 0]
    a_d = ad_ref[0, 0]
    as0 = (h0 * a_s[0][None, :]).sum(-1)
    as1 = (h1 * a_s[1][None, :]).sum(-1)
    ad0 = (h0 * a_d[0][None, :]).sum(-1)
    ad1 = (h1 * a_d[1][None, :]).sum(-1)
    a_ref[0] = jnp.stack(
        [jnp.stack([as0, ad0], axis=-1), jnp.stack([as1, ad1], axis=-1)],
        axis=1)


def _run_k1(x0, W1, a_src1, a_dst1):
    return pl.pallas_call(
        _k1_body,
        grid=(NREL, N // TN),
        in_specs=[
            pl.BlockSpec((TN, EMB), lambda r, i: (i, 0)),
            pl.BlockSpec((1, EMB, HEADS * HID), lambda r, i: (r, 0, 0)),
            pl.BlockSpec((1, 1, HEADS, HID), lambda r, i: (r, 0, 0, 0)),
            pl.BlockSpec((1, 1, HEADS, HID), lambda r, i: (r, 0, 0, 0)),
        ],
        out_specs=[
            pl.BlockSpec((1, TN, HID), lambda r, i: (r, i, 0)),
            pl.BlockSpec((1, TN, HID), lambda r, i: (r, i, 0)),
            pl.BlockSpec((1, TN, 2, 2), lambda r, i: (r, i, 0, 0)),
        ],
        out_shape=[
            jax.ShapeDtypeStruct((NREL, N, HID), _f32),
            jax.ShapeDtypeStruct((NREL, N, HID), _f32),
            jax.ShapeDtypeStruct((NREL, N, 2, 2), _f32),
        ],
    )(x0, W1, a_src1, a_dst1)


def _post1_body(acc_ref, h0_ref, h1_ref, a_ref, den_ref, b_ref, x_ref):
    at = a_ref[...]
    dt = den_ref[...]
    out = jnp.zeros((at.shape[0], HEADS * HID), _f32)
    for r in range(NREL):
        halves = []
        for c in range(2):
            col = (r * 2 + c) * 2
            asr = at[:, col]
            ads = at[:, col + 1]
            al = asr + ads
            el = jnp.exp(jnp.where(al >= 0.0, al, al * 0.2))
            den = dt[:, r * 2 + c] + el + 1e-16
            hrc = h0_ref[r] if c == 0 else h1_ref[r]
            num = acc_ref[r, c] + hrc * el[:, None]
            halves.append(num / den[:, None])
        row = jnp.concatenate(halves, axis=-1) + b_ref[r][None, :]
        out = out + jnp.where(row > 0.0, row, jnp.exp(row) - 1.0)
    x_ref[...] = out


def _run_post1(acc, h0, h1, a_t, den_t, b1):
    return pl.pallas_call(
        _post1_body,
        grid=(N // TN,),
        in_specs=[
            pl.BlockSpec((NREL, 2, TN, HID), lambda i: (0, 0, i, 0)),
            pl.BlockSpec((NREL, TN, HID), lambda i: (0, i, 0)),
            pl.BlockSpec((NREL, TN, HID), lambda i: (0, i, 0)),
            pl.BlockSpec((TN, 16), lambda i: (i, 0)),
            pl.BlockSpec((TN, 8), lambda i: (i, 0)),
            pl.BlockSpec((NREL, HEADS * HID), lambda i: (0, 0)),
        ],
        out_specs=pl.BlockSpec((TN, HEADS * HID), lambda i: (i, 0)),
        out_shape=jax.ShapeDtypeStruct((N, HEADS * HID), _f32),
    )(acc, h0, h1, a_t, den_t, b1)


def _k2_body(x_ref, w_ref, as_ref, ad_ref, h0_ref, h1_ref, a_ref):
    xb = x_ref[...]
    h = jnp.dot(xb, w_ref[0], preferred_element_type=_f32)
    h0_ref[0] = h[:, :OUT // 2]
    h1_ref[0] = h[:, OUT // 2:]
    asr = (h * as_ref[0, 0, 0][None, :]).sum(-1)
    ads = (h * ad_ref[0, 0, 0][None, :]).sum(-1)
    a_ref[0] = jnp.stack([asr, ads], axis=-1)


def _run_k2(x, W2, a_src2, a_dst2):
    return pl.pallas_call(
        _k2_body,
        grid=(NREL, N // TN),
        in_specs=[
            pl.BlockSpec((TN, HEADS * HID), lambda r, i: (i, 0)),
            pl.BlockSpec((1, HEADS * HID, OUT), lambda r, i: (r, 0, 0)),
            pl.BlockSpec((1, 1, 1, OUT), lambda r, i: (r, 0, 0, 0)),
            pl.BlockSpec((1, 1, 1, OUT), lambda r, i: (r, 0, 0, 0)),
        ],
        out_specs=[
            pl.BlockSpec((1, TN, OUT // 2), lambda r, i: (r, i, 0)),
            pl.BlockSpec((1, TN, OUT // 2), lambda r, i: (r, i, 0)),
            pl.BlockSpec((1, TN, 2), lambda r, i: (r, i, 0)),
        ],
        out_shape=[
            jax.ShapeDtypeStruct((NREL, N, OUT // 2), _f32),
            jax.ShapeDtypeStruct((NREL, N, OUT // 2), _f32),
            jax.ShapeDtypeStruct((NREL, N, 2), _f32),
        ],
    )(x, W2, a_src2, a_dst2)


def _post2_body(acc_ref, h0_ref, h1_ref, a_ref, den_ref, b_ref, x0_ref,
                rw_ref, rb_ref, g_ref, bb_ref, ent_ref):
    at = a_ref[...]
    dt = den_ref[...]
    x2 = jnp.zeros((at.shape[0], OUT), _f32)
    for r in range(NREL):
        asr = at[:, 2 * r]
        ads = at[:, 2 * r + 1]
        al = asr + ads
        el = jnp.exp(jnp.where(al >= 0.0, al, al * 0.2))
        den = dt[:, r] + el + 1e-16
        num = jnp.concatenate(
            [acc_ref[r, 0] + h0_ref[r] * el[:, None],
             acc_ref[r, 1] + h1_ref[r] * el[:, None]], axis=-1)
        x2 = x2 + num / den[:, None] + b_ref[r][None, :]
    pre = x2 + jnp.dot(x0_ref[...], rw_ref[...],
                       preferred_element_type=_f32) + rb_ref[...][None, :]
    mu = pre.mean(axis=-1, keepdims=True)
    d = pre - mu
    var = (d * d).mean(axis=-1, keepdims=True)
    ent_ref[...] = d * lax.rsqrt(var + 1e-5) * g_ref[...][None, :] \
        + bb_ref[...][None, :]


def _run_post2(acc, h0, h1, a_t, den_t, b2, x0, res_W, res_b, ln_g, ln_b):
    return pl.pallas_call(
        _post2_body,
        grid=(N // TN,),
        in_specs=[
            pl.BlockSpec((NREL, 2, TN, OUT // 2), lambda i: (0, 0, i, 0)),
            pl.BlockSpec((NREL, TN, OUT // 2), lambda i: (0, i, 0)),
            pl.BlockSpec((NREL, TN, OUT // 2), lambda i: (0, i, 0)),
            pl.BlockSpec((TN, 8), lambda i: (i, 0)),
            pl.BlockSpec((TN, 4), lambda i: (i, 0)),
            pl.BlockSpec((NREL, OUT), lambda i: (0, 0)),
            pl.BlockSpec((TN, EMB), lambda i: (i, 0)),
            pl.BlockSpec((EMB, OUT), lambda i: (0, 0)),
            pl.BlockSpec((OUT,), lambda i: (0,)),
            pl.BlockSpec((OUT,), lambda i: (0,)),
            pl.BlockSpec((OUT,), lambda i: (0,)),
        ],
        out_specs=pl.BlockSpec((TN, OUT), lambda i: (i, 0)),
        out_shape=jax.ShapeDtypeStruct((N, OUT), _f32),
    )(acc, h0, h1, a_t, den_t, b2, x0, res_W, res_b, ln_g, ln_b)


def kernel(triples, edge_index, entity_emb, W1, a_src1, a_dst1, b1, W2,
           a_src2, a_dst2, b2, res_W, res_b, ln_g, ln_b, rel_emb):
    x0 = entity_emb
    edge_index = edge_index.astype(_i32)

    h1c0, h1c1, aout1 = _run_k1(x0, W1, a_src1, a_dst1)

    edge_fn1 = _make_edge_kernel(HID)
    accs1, dens1 = [], []
    for r in range(NREL):
        as_hm = aout1[r, :, :, 0].T.reshape(2 * N)
        ad_hm = aout1[r, :, :, 1].T.reshape(2 * N)
        acc, den = edge_fn1(edge_index[r, 0], edge_index[r, 1],
                            h1c0[r], h1c1[r], as_hm, ad_hm)
        accs1.append(acc)
        dens1.append(den)
    acc1 = jnp.stack(accs1).reshape(NREL, 2, N, HID)
    a1t = aout1.transpose(1, 0, 2, 3).reshape(N, 16)
    den1t = jnp.stack(dens1).transpose(2, 0, 1).reshape(N, 8)

    x = _run_post1(acc1, h1c0, h1c1, a1t, den1t, b1)

    h2c0, h2c1, aout2 = _run_k2(x, W2, a_src2, a_dst2)

    edge_fn2 = _make_edge_kernel(OUT // 2)
    accs2, dens2 = [], []
    for r in range(NREL):
        a_flat = jnp.concatenate([aout2[r, :, 0], aout2[r, :, 0]])
        d_flat = jnp.concatenate([aout2[r, :, 1], aout2[r, :, 1]])
        acc, den = edge_fn2(edge_index[r, 0], edge_index[r, 1],
                            h2c0[r], h2c1[r], a_flat, d_flat)
        accs2.append(acc)
        dens2.append(den)
    acc2 = jnp.stack(accs2).reshape(NREL, 2, N, OUT // 2)
    a2t = aout2.transpose(1, 0, 2).reshape(N, 8)
    den2t = jnp.stack(dens2)[:, 0].transpose(1, 0)

    ent = _run_post2(acc2, h2c0, h2c1, a2t, den2t, b2, x0, res_W, res_b,
                     ln_g, ln_b)

    wrow = _run_wsel(triples[:, 1:2].astype(_i32), rel_emb)
    score_fn = _make_score_kernel()
    score = score_fn(triples[:, 0].astype(_i32), triples[:, 2].astype(_i32),
                     wrow, ent)
    return score


# speedup vs baseline: 35.7739x; 1.0289x over previous
import functools

import jax
import jax.numpy as jnp
from jax import lax
from jax.experimental import pallas as pl
from jax.experimental.pallas import tpu as pltpu
from jax.experimental.pallas import tpu_sc as plsc

N = 10000
NREL = 4
E = 80000
EMB = 128
HID = 64
HEADS = 2
OUT = 256
B = 16384

NCHUNK = E // 128
MAXCH = (NCHUNK + 15) // 16
TN = 1000
DEN_PAD = 10240

_i32 = jnp.int32
_f32 = jnp.float32


def _edge_body(Fh, src_hbm, dst_hbm, h0_hbm, h1_hbm, as0_hbm, as1_hbm,
               ad0_hbm, ad1_hbm,
               acc_hbm, den_hbm,
               srcbA, dstbA, asbA, adbA, echA, gbufA,
               srcbB, dstbB, asbB, adbB, echB, gbufB,
               zbuf, den_sh, acc_sh, gsemA, gsemB, asemA, asemB):
    c = lax.axis_index("c")
    s = lax.axis_index("s")
    base = c * N
    zero16 = jnp.zeros((16,), _f32)

    def _zg(i, carry):
        for f in range(Fh // 16):
            gbufA[i, pl.ds(f * 16, 16)] = zero16
        return carry
    lax.fori_loop(0, 128, _zg, 0)
    def _zz(i, carry):
        zbuf[pl.ds(i * 16, 16)] = zero16
        return carry
    lax.fori_loop(0, 40, _zz, 0)

    r0 = s * 625
    for kk in range(4):
        pltpu.sync_copy(gbufA, acc_sh.at[pl.ds(r0 + kk * 128, 128)])
    pltpu.sync_copy(gbufA.at[pl.ds(0, 113)], acc_sh.at[pl.ds(r0 + 512, 113)])
    pltpu.sync_copy(zbuf, den_sh.at[pl.ds(s * 640, 640)])

    plsc.subcore_barrier()

    def _prefetch(ci, srcb, dstb, asb, adb, gbuf, gsem, asem):
        pltpu.sync_copy(src_hbm.at[pl.ds(ci * 128, 128)], srcb)
        pltpu.sync_copy(dst_hbm.at[pl.ds(ci * 128, 128)], dstb)

        @pl.when(c == 0)
        def _():
            pltpu.async_copy(h0_hbm.at[srcb], gbuf, gsem)
            pltpu.async_copy(as0_hbm.at[srcb], asb, asem)
            pltpu.async_copy(ad0_hbm.at[dstb], adb, asem)

        @pl.when(c == 1)
        def _():
            pltpu.async_copy(h1_hbm.at[srcb], gbuf, gsem)
            pltpu.async_copy(as1_hbm.at[srcb], asb, asem)
            pltpu.async_copy(ad1_hbm.at[dstb], adb, asem)

    def _process(srcb, dstb, asb, adb, ech, gbuf, gsem, asem):
        pltpu.make_async_copy(as0_hbm.at[srcb], asb, asem).wait()
        pltpu.make_async_copy(ad0_hbm.at[dstb], adb, asem).wait()
        for g in range(8):
            av = asb[pl.ds(g * 16, 16)] + adb[pl.ds(g * 16, 16)]
            av = jnp.where(av >= 0.0, av, av * jnp.float32(0.2))
            ech[pl.ds(g * 16, 16)] = jnp.exp(av)

        pltpu.sync_copy(ech, den_sh.at[dstb], add=True)

        pltpu.make_async_copy(h0_hbm.at[srcb], gbuf, gsem).wait()

        def _scale(rw, carry2):
            r2 = 2 * rw
            ev = plsc.load_gather(ech, [jnp.full((16,), r2, _i32)])
            ev2 = plsc.load_gather(ech, [jnp.full((16,), r2 + 1, _i32)])
            for f in range(Fh // 16):
                gbuf[r2, pl.ds(f * 16, 16)] = (
                    gbuf[r2, pl.ds(f * 16, 16)] * ev)
            for f in range(Fh // 16):
                gbuf[r2 + 1, pl.ds(f * 16, 16)] = (
                    gbuf[r2 + 1, pl.ds(f * 16, 16)] * ev2)
            return carry2
        lax.fori_loop(0, 64, _scale, 0)

        pltpu.sync_copy(gbuf, acc_sh.at[dstb], add=True)

    _prefetch(s, srcbA, dstbA, asbA, adbA, gbufA, gsemA, asemA)

    def _chunk(j, carry):
        ci = s + 16 * j

        @pl.when(((j & 1) == 0) & (ci < NCHUNK))
        def _():
            @pl.when(ci + 16 < NCHUNK)
            def _():
                _prefetch(ci + 16, srcbB, dstbB, asbB, adbB, gbufB,
                          gsemB, asemB)
            _process(srcbA, dstbA, asbA, adbA, echA, gbufA, gsemA, asemA)

        @pl.when(((j & 1) == 1) & (ci < NCHUNK))
        def _():
            @pl.when(ci + 16 < NCHUNK)
            def _():
                _prefetch(ci + 16, srcbA, dstbA, asbA, adbA, gbufA,
                          gsemA, asemA)
            _process(srcbB, dstbB, asbB, adbB, echB, gbufB, gsemB, asemB)
        return carry
    lax.fori_loop(0, MAXCH, _chunk, 0)

    plsc.subcore_barrier()

    pltpu.sync_copy(acc_sh.at[pl.ds(r0, 625)],
                    acc_hbm.at[pl.ds(base + r0, 625)])

    @pl.when(s == 0)
    def _():
        pltpu.sync_copy(den_sh.at[pl.ds(0, N)], den_hbm.at[c])


@functools.cache
def _make_edge_kernel(Fh):
    mesh = plsc.VectorSubcoreMesh(core_axis_name="c", subcore_axis_name="s")
    return pl.kernel(
        functools.partial(_edge_body, Fh),
        out_type=[
            jax.ShapeDtypeStruct((2 * N, Fh), _f32),
            jax.ShapeDtypeStruct((2, N), _f32),
        ],
        mesh=mesh,
        scratch_types=[
            pltpu.VMEM((128,), _i32),
            pltpu.VMEM((128,), _i32),
            pltpu.VMEM((128,), _f32),
            pltpu.VMEM((128,), _f32),
            pltpu.VMEM((128,), _f32),
            pltpu.VMEM((128, Fh), _f32),
            pltpu.VMEM((128,), _i32),
            pltpu.VMEM((128,), _i32),
            pltpu.VMEM((128,), _f32),
            pltpu.VMEM((128,), _f32),
            pltpu.VMEM((128,), _f32),
            pltpu.VMEM((128, Fh), _f32),
            pltpu.VMEM((640,), _f32),
            pltpu.VMEM_SHARED((DEN_PAD,), _f32),
            pltpu.VMEM_SHARED((N, Fh), _f32),
            pltpu.SemaphoreType.DMA,
            pltpu.SemaphoreType.DMA,
            pltpu.SemaphoreType.DMA,
            pltpu.SemaphoreType.DMA,
        ],
        compiler_params=pltpu.CompilerParams(use_tc_tiling_on_sc=False, needs_layout_passes=False),
    )


def _score_body(hcol_hbm, tcol_hbm, wrow_hbm, ent_hbm, out_hbm,
                hl, tl, ehb, etb, wrb, pb, sb, sem1, sem2):
    c = lax.axis_index("c")
    s = lax.axis_index("s")
    w = c * 16 + s
    off = w * 512

    pltpu.sync_copy(hcol_hbm.at[pl.ds(off, 512)], hl)
    pltpu.sync_copy(tcol_hbm.at[pl.ds(off, 512)], tl)

    def _chunk(q, carry):
        cp1 = pltpu.async_copy(ent_hbm.at[hl.at[pl.ds(q * 64, 64)]], ehb, sem1)
        cp2 = pltpu.async_copy(ent_hbm.at[tl.at[pl.ds(q * 64, 64)]], etb, sem2)
        pltpu.sync_copy(wrow_hbm.at[pl.ds(off + q * 64, 64)], wrb)
        cp1.wait()
        cp2.wait()

        def _triple(j, carry2):
            acc = jnp.zeros((16,), _f32)
            for f in range(OUT // 16):
                acc = acc + (ehb[j, pl.ds(f * 16, 16)]
                             * etb[j, pl.ds(f * 16, 16)]
                             * wrb[j, pl.ds(f * 16, 16)])
            pb[q * 64 + j] = acc
            return carry2
        lax.fori_loop(0, 64, _triple, 0)
        return carry
    lax.fori_loop(0, 8, _chunk, 0)

    iota16 = lax.iota(_i32, 16)

    def _red(jg, carry):
        rows = iota16 + jg * 16
        tot = jnp.zeros((16,), _f32)
        for f in range(16):
            tot = tot + plsc.load_gather(pb, [rows, jnp.full((16,), f, _i32)])
        sb[pl.ds(jg * 16, 16)] = tot
        return carry
    lax.fori_loop(0, 32, _red, 0)

    pltpu.sync_copy(sb, out_hbm.at[pl.ds(off, 512)])


@functools.cache
def _make_score_kernel():
    mesh = plsc.VectorSubcoreMesh(core_axis_name="c", subcore_axis_name="s")
    return pl.kernel(
        _score_body,
        out_type=jax.ShapeDtypeStruct((B,), _f32),
        mesh=mesh,
        scratch_types=[
            pltpu.VMEM((512,), _i32),
            pltpu.VMEM((512,), _i32),
            pltpu.VMEM((64, OUT), _f32),
            pltpu.VMEM((64, OUT), _f32),
            pltpu.VMEM((64, OUT), _f32),
            pltpu.VMEM((512, 16), _f32),
            pltpu.VMEM((512,), _f32),
            pltpu.SemaphoreType.DMA,
            pltpu.SemaphoreType.DMA,
        ],
        compiler_params=pltpu.CompilerParams(use_tc_tiling_on_sc=False, needs_layout_passes=False),
    )


def _wsel_body(rc_ref, emb_ref, o_ref):
    rc = rc_ref[...]
    out = jnp.zeros((rc.shape[0], OUT), _f32)
    for r in range(NREL):
        out = out + jnp.where(rc == r, 1.0, 0.0) * emb_ref[r][None, :]
    o_ref[...] = out


def _run_wsel(rcol2d, rel_emb):
    TB = 2048
    return pl.pallas_call(
        _wsel_body,
        grid=(B // TB,),
        in_specs=[
            pl.BlockSpec((TB, 1), lambda i: (i, 0)),
            pl.BlockSpec((NREL, OUT), lambda i: (0, 0)),
        ],
        out_specs=pl.BlockSpec((TB, OUT), lambda i: (i, 0)),
        out_shape=jax.ShapeDtypeStruct((B, OUT), _f32),
    )(rcol2d, rel_emb)


def _k1_body(x_ref, w_ref, as_ref, ad_ref, h0_ref, h1_ref, a_ref):
    xb = x_ref[...]
    h = jnp.dot(xb, w_ref[0], preferred_element_type=_f32)
    h0 = h[:, :HID]
    h1 = h[:, HID:]
    h0_ref[0] = h0
    h1_ref[0] = h1
    a_s = as_ref[0, 0]
    a_d = ad_ref[0, 0]
    as0 = (h0 * a_s[0][None, :]).sum(-1)
    as1 = (h1 * a_s[1][None, :]).sum(-1)
    ad0 = (h0 * a_d[0][None, :]).sum(-1)
    ad1 = (h1 * a_d[1][None, :]).sum(-1)
    a_ref[0] = jnp.stack(
        [jnp.stack([as0, ad0], axis=-1), jnp.stack([as1, ad1], axis=-1)],
        axis=1)


def _run_k1(x0, W1, a_src1, a_dst1):
    return pl.pallas_call(
        _k1_body,
        grid=(NREL, N // TN),
        in_specs=[
            pl.BlockSpec((TN, EMB), lambda r, i: (i, 0)),
            pl.BlockSpec((1, EMB, HEADS * HID), lambda r, i: (r, 0, 0)),
            pl.BlockSpec((1, 1, HEADS, HID), lambda r, i: (r, 0, 0, 0)),
            pl.BlockSpec((1, 1, HEADS, HID), lambda r, i: (r, 0, 0, 0)),
        ],
        out_specs=[
            pl.BlockSpec((1, TN, HID), lambda r, i: (r, i, 0)),
            pl.BlockSpec((1, TN, HID), lambda r, i: (r, i, 0)),
            pl.BlockSpec((1, TN, 2, 2), lambda r, i: (r, i, 0, 0)),
        ],
        out_shape=[
            jax.ShapeDtypeStruct((NREL, N, HID), _f32),
            jax.ShapeDtypeStruct((NREL, N, HID), _f32),
            jax.ShapeDtypeStruct((NREL, N, 2, 2), _f32),
        ],
    )(x0, W1, a_src1, a_dst1)


def _post1_body(acc_ref, h0_ref, h1_ref, a_ref, den_ref, b_ref, x_ref):
    at = a_ref[...]
    dt = den_ref[...]
    out = jnp.zeros((at.shape[0], HEADS * HID), _f32)
    for r in range(NREL):
        halves = []
        for c in range(2):
            col = (r * 2 + c) * 2
            asr = at[:, col]
            ads = at[:, col + 1]
            al = asr + ads
            el = jnp.exp(jnp.where(al >= 0.0, al, al * 0.2))
            den = dt[:, r * 2 + c] + el + 1e-16
            hrc = h0_ref[r] if c == 0 else h1_ref[r]
            num = acc_ref[r, c] + hrc * el[:, None]
            halves.append(num / den[:, None])
        row = jnp.concatenate(halves, axis=-1) + b_ref[r][None, :]
        out = out + jnp.where(row > 0.0, row, jnp.exp(row) - 1.0)
    x_ref[...] = out


def _run_post1(acc, h0, h1, a_t, den_t, b1):
    return pl.pallas_call(
        _post1_body,
        grid=(N // TN,),
        in_specs=[
            pl.BlockSpec((NREL, 2, TN, HID), lambda i: (0, 0, i, 0)),
            pl.BlockSpec((NREL, TN, HID), lambda i: (0, i, 0)),
            pl.BlockSpec((NREL, TN, HID), lambda i: (0, i, 0)),
            pl.BlockSpec((TN, 16), lambda i: (i, 0)),
            pl.BlockSpec((TN, 8), lambda i: (i, 0)),
            pl.BlockSpec((NREL, HEADS * HID), lambda i: (0, 0)),
        ],
        out_specs=pl.BlockSpec((TN, HEADS * HID), lambda i: (i, 0)),
        out_shape=jax.ShapeDtypeStruct((N, HEADS * HID), _f32),
    )(acc, h0, h1, a_t, den_t, b1)


def _k2_body(x_ref, w_ref, as_ref, ad_ref, h0_ref, h1_ref, a_ref):
    xb = x_ref[...]
    h = jnp.dot(xb, w_ref[0], preferred_element_type=_f32)
    h0_ref[0] = h[:, :OUT // 2]
    h1_ref[0] = h[:, OUT // 2:]
    asr = (h * as_ref[0, 0, 0][None, :]).sum(-1)
    ads = (h * ad_ref[0, 0, 0][None, :]).sum(-1)
    a_ref[0] = jnp.stack([asr, ads], axis=-1)


def _run_k2(x, W2, a_src2, a_dst2):
    return pl.pallas_call(
        _k2_body,
        grid=(NREL, N // TN),
        in_specs=[
            pl.BlockSpec((TN, HEADS * HID), lambda r, i: (i, 0)),
            pl.BlockSpec((1, HEADS * HID, OUT), lambda r, i: (r, 0, 0)),
            pl.BlockSpec((1, 1, 1, OUT), lambda r, i: (r, 0, 0, 0)),
            pl.BlockSpec((1, 1, 1, OUT), lambda r, i: (r, 0, 0, 0)),
        ],
        out_specs=[
            pl.BlockSpec((1, TN, OUT // 2), lambda r, i: (r, i, 0)),
            pl.BlockSpec((1, TN, OUT // 2), lambda r, i: (r, i, 0)),
            pl.BlockSpec((1, TN, 2), lambda r, i: (r, i, 0)),
        ],
        out_shape=[
            jax.ShapeDtypeStruct((NREL, N, OUT // 2), _f32),
            jax.ShapeDtypeStruct((NREL, N, OUT // 2), _f32),
            jax.ShapeDtypeStruct((NREL, N, 2), _f32),
        ],
    )(x, W2, a_src2, a_dst2)


def _post2_body(acc_ref, h0_ref, h1_ref, a_ref, den_ref, b_ref, x0_ref,
                rw_ref, rb_ref, g_ref, bb_ref, ent_ref):
    at = a_ref[...]
    dt = den_ref[...]
    x2 = jnp.zeros((at.shape[0], OUT), _f32)
    for r in range(NREL):
        asr = at[:, 2 * r]
        ads = at[:, 2 * r + 1]
        al = asr + ads
        el = jnp.exp(jnp.where(al >= 0.0, al, al * 0.2))
        den = dt[:, r] + el + 1e-16
        num = jnp.concatenate(
            [acc_ref[r, 0] + h0_ref[r] * el[:, None],
             acc_ref[r, 1] + h1_ref[r] * el[:, None]], axis=-1)
        x2 = x2 + num / den[:, None] + b_ref[r][None, :]
    pre = x2 + jnp.dot(x0_ref[...], rw_ref[...],
                       preferred_element_type=_f32) + rb_ref[...][None, :]
    mu = pre.mean(axis=-1, keepdims=True)
    d = pre - mu
    var = (d * d).mean(axis=-1, keepdims=True)
    ent_ref[...] = d * lax.rsqrt(var + 1e-5) * g_ref[...][None, :] \
        + bb_ref[...][None, :]


def _run_post2(acc, h0, h1, a_t, den_t, b2, x0, res_W, res_b, ln_g, ln_b):
    return pl.pallas_call(
        _post2_body,
        grid=(N // TN,),
        in_specs=[
            pl.BlockSpec((NREL, 2, TN, OUT // 2), lambda i: (0, 0, i, 0)),
            pl.BlockSpec((NREL, TN, OUT // 2), lambda i: (0, i, 0)),
            pl.BlockSpec((NREL, TN, OUT // 2), lambda i: (0, i, 0)),
            pl.BlockSpec((TN, 8), lambda i: (i, 0)),
            pl.BlockSpec((TN, 4), lambda i: (i, 0)),
            pl.BlockSpec((NREL, OUT), lambda i: (0, 0)),
            pl.BlockSpec((TN, EMB), lambda i: (i, 0)),
            pl.BlockSpec((EMB, OUT), lambda i: (0, 0)),
            pl.BlockSpec((OUT,), lambda i: (0,)),
            pl.BlockSpec((OUT,), lambda i: (0,)),
            pl.BlockSpec((OUT,), lambda i: (0,)),
        ],
        out_specs=pl.BlockSpec((TN, OUT), lambda i: (i, 0)),
        out_shape=jax.ShapeDtypeStruct((N, OUT), _f32),
    )(acc, h0, h1, a_t, den_t, b2, x0, res_W, res_b, ln_g, ln_b)


def kernel(triples, edge_index, entity_emb, W1, a_src1, a_dst1, b1, W2,
           a_src2, a_dst2, b2, res_W, res_b, ln_g, ln_b, rel_emb):
    x0 = entity_emb
    edge_index = edge_index.astype(_i32)

    h1c0, h1c1, aout1 = _run_k1(x0, W1, a_src1, a_dst1)

    edge_fn1 = _make_edge_kernel(HID)
    accs1, dens1 = [], []
    for r in range(NREL):
        acc, den = edge_fn1(edge_index[r, 0], edge_index[r, 1],
                            h1c0[r], h1c1[r],
                            aout1[r, :, 0, 0], aout1[r, :, 1, 0],
                            aout1[r, :, 0, 1], aout1[r, :, 1, 1])
        accs1.append(acc)
        dens1.append(den)
    acc1 = jnp.stack(accs1).reshape(NREL, 2, N, HID)
    a1t = aout1.transpose(1, 0, 2, 3).reshape(N, 16)
    den1t = jnp.stack(dens1).transpose(2, 0, 1).reshape(N, 8)

    x = _run_post1(acc1, h1c0, h1c1, a1t, den1t, b1)

    h2c0, h2c1, aout2 = _run_k2(x, W2, a_src2, a_dst2)

    edge_fn2 = _make_edge_kernel(OUT // 2)
    accs2, dens2 = [], []
    for r in range(NREL):
        acc, den = edge_fn2(edge_index[r, 0], edge_index[r, 1],
                            h2c0[r], h2c1[r],
                            aout2[r, :, 0], aout2[r, :, 0],
                            aout2[r, :, 1], aout2[r, :, 1])
        accs2.append(acc)
        dens2.append(den)
    acc2 = jnp.stack(accs2).reshape(NREL, 2, N, OUT // 2)
    a2t = aout2.transpose(1, 0, 2).reshape(N, 8)
    den2t = jnp.stack(dens2)[:, 0].transpose(1, 0)

    ent = _run_post2(acc2, h2c0, h2c1, a2t, den2t, b2, x0, res_W, res_b,
                     ln_g, ln_b)

    wrow = _run_wsel(triples[:, 1:2].astype(_i32), rel_emb)
    score_fn = _make_score_kernel()
    score = score_fn(triples[:, 0].astype(_i32), triples[:, 2].astype(_i32),
                     wrow, ent)
    return score


# speedup vs baseline: 37.2618x; 1.0416x over previous
import functools

import jax
import jax.numpy as jnp
from jax import lax
from jax.experimental import pallas as pl
from jax.experimental.pallas import tpu as pltpu
from jax.experimental.pallas import tpu_sc as plsc

N = 10000
NREL = 4
E = 80000
EMB = 128
HID = 64
HEADS = 2
OUT = 256
B = 16384

NCHUNK = E // 128
MAXCH = (NCHUNK + 15) // 16
TN = 1000
DEN_PAD = 10240

_i32 = jnp.int32
_f32 = jnp.float32


def _edge_body(Fh, src_hbm, dst_hbm, h0_hbm, h1_hbm, as0_hbm, as1_hbm,
               ad0_hbm, ad1_hbm,
               acc_hbm, den_hbm,
               srcbA, dstbA, asbA, adbA, echA, gbufA,
               srcbB, dstbB, asbB, adbB, echB, gbufB,
               zbuf, den_sh, acc_sh, gsemA, gsemB, asemA, asemB):
    c = lax.axis_index("c")
    s = lax.axis_index("s")
    base = c * N
    zero16 = jnp.zeros((16,), _f32)

    def _zg(i, carry):
        for f in range(Fh // 16):
            gbufA[i, pl.ds(f * 16, 16)] = zero16
        return carry
    lax.fori_loop(0, 128, _zg, 0)
    def _zz(i, carry):
        zbuf[pl.ds(i * 16, 16)] = zero16
        return carry
    lax.fori_loop(0, 40, _zz, 0)

    r0 = s * 625
    for kk in range(4):
        pltpu.sync_copy(gbufA, acc_sh.at[pl.ds(r0 + kk * 128, 128)])
    pltpu.sync_copy(gbufA.at[pl.ds(0, 113)], acc_sh.at[pl.ds(r0 + 512, 113)])
    pltpu.sync_copy(zbuf, den_sh.at[pl.ds(s * 640, 640)])

    plsc.subcore_barrier()

    def _prefetch(ci, srcb, dstb, asb, adb, gbuf, gsem, asem):
        pltpu.sync_copy(src_hbm.at[pl.ds(ci * 128, 128)], srcb)
        pltpu.sync_copy(dst_hbm.at[pl.ds(ci * 128, 128)], dstb)

        @pl.when(c == 0)
        def _():
            pltpu.async_copy(h0_hbm.at[srcb], gbuf, gsem)
            pltpu.async_copy(as0_hbm.at[srcb], asb, asem)
            pltpu.async_copy(ad0_hbm.at[dstb], adb, asem)

        @pl.when(c == 1)
        def _():
            pltpu.async_copy(h1_hbm.at[srcb], gbuf, gsem)
            pltpu.async_copy(as1_hbm.at[srcb], asb, asem)
            pltpu.async_copy(ad1_hbm.at[dstb], adb, asem)

    def _process(srcb, dstb, asb, adb, ech, gbuf, gsem, asem):
        pltpu.make_async_copy(as0_hbm.at[srcb], asb, asem).wait()
        pltpu.make_async_copy(ad0_hbm.at[dstb], adb, asem).wait()
        for g in range(8):
            av = asb[pl.ds(g * 16, 16)] + adb[pl.ds(g * 16, 16)]
            av = jnp.where(av >= 0.0, av, av * jnp.float32(0.2))
            ech[pl.ds(g * 16, 16)] = jnp.exp(av)

        pltpu.sync_copy(ech, den_sh.at[dstb], add=True)

        pltpu.make_async_copy(h0_hbm.at[srcb], gbuf, gsem).wait()

        @plsc.parallel_loop(0, 128, step=2, unroll=4)
        def _scale(rw):
            ev = plsc.load_gather(ech, [jnp.full((16,), rw, _i32)])
            ev2 = plsc.load_gather(ech, [jnp.full((16,), rw + 1, _i32)])
            for f in range(Fh // 16):
                gbuf[rw, pl.ds(f * 16, 16)] = (
                    gbuf[rw, pl.ds(f * 16, 16)] * ev)
            for f in range(Fh // 16):
                gbuf[rw + 1, pl.ds(f * 16, 16)] = (
                    gbuf[rw + 1, pl.ds(f * 16, 16)] * ev2)

        pltpu.sync_copy(gbuf, acc_sh.at[dstb], add=True)

    _prefetch(s, srcbA, dstbA, asbA, adbA, gbufA, gsemA, asemA)

    def _chunk(j, carry):
        ci = s + 16 * j

        @pl.when(((j & 1) == 0) & (ci < NCHUNK))
        def _():
            @pl.when(ci + 16 < NCHUNK)
            def _():
                _prefetch(ci + 16, srcbB, dstbB, asbB, adbB, gbufB,
                          gsemB, asemB)
            _process(srcbA, dstbA, asbA, adbA, echA, gbufA, gsemA, asemA)

        @pl.when(((j & 1) == 1) & (ci < NCHUNK))
        def _():
            @pl.when(ci + 16 < NCHUNK)
            def _():
                _prefetch(ci + 16, srcbA, dstbA, asbA, adbA, gbufA,
                          gsemA, asemA)
            _process(srcbB, dstbB, asbB, adbB, echB, gbufB, gsemB, asemB)
        return carry
    lax.fori_loop(0, MAXCH, _chunk, 0)

    plsc.subcore_barrier()

    pltpu.sync_copy(acc_sh.at[pl.ds(r0, 625)],
                    acc_hbm.at[pl.ds(base + r0, 625)])

    @pl.when(s == 0)
    def _():
        pltpu.sync_copy(den_sh.at[pl.ds(0, N)], den_hbm.at[c])


@functools.cache
def _make_edge_kernel(Fh):
    mesh = plsc.VectorSubcoreMesh(core_axis_name="c", subcore_axis_name="s")
    return pl.kernel(
        functools.partial(_edge_body, Fh),
        out_type=[
            jax.ShapeDtypeStruct((2 * N, Fh), _f32),
            jax.ShapeDtypeStruct((2, N), _f32),
        ],
        mesh=mesh,
        scratch_types=[
            pltpu.VMEM((128,), _i32),
            pltpu.VMEM((128,), _i32),
            pltpu.VMEM((128,), _f32),
            pltpu.VMEM((128,), _f32),
            pltpu.VMEM((128,), _f32),
            pltpu.VMEM((128, Fh), _f32),
            pltpu.VMEM((128,), _i32),
            pltpu.VMEM((128,), _i32),
            pltpu.VMEM((128,), _f32),
            pltpu.VMEM((128,), _f32),
            pltpu.VMEM((128,), _f32),
            pltpu.VMEM((128, Fh), _f32),
            pltpu.VMEM((640,), _f32),
            pltpu.VMEM_SHARED((DEN_PAD,), _f32),
            pltpu.VMEM_SHARED((N, Fh), _f32),
            pltpu.SemaphoreType.DMA,
            pltpu.SemaphoreType.DMA,
            pltpu.SemaphoreType.DMA,
            pltpu.SemaphoreType.DMA,
        ],
        compiler_params=pltpu.CompilerParams(use_tc_tiling_on_sc=False, needs_layout_passes=False),
    )


def _score_body(hcol_hbm, tcol_hbm, wrow_hbm, ent_hbm, out_hbm,
                hl, tl, ehb, etb, wrb, pb, sb, sem1, sem2):
    c = lax.axis_index("c")
    s = lax.axis_index("s")
    w = c * 16 + s
    off = w * 512

    pltpu.sync_copy(hcol_hbm.at[pl.ds(off, 512)], hl)
    pltpu.sync_copy(tcol_hbm.at[pl.ds(off, 512)], tl)

    def _chunk(q, carry):
        cp1 = pltpu.async_copy(ent_hbm.at[hl.at[pl.ds(q * 64, 64)]], ehb, sem1)
        cp2 = pltpu.async_copy(ent_hbm.at[tl.at[pl.ds(q * 64, 64)]], etb, sem2)
        pltpu.sync_copy(wrow_hbm.at[pl.ds(off + q * 64, 64)], wrb)
        cp1.wait()
        cp2.wait()

        def _triple(j, carry2):
            acc = jnp.zeros((16,), _f32)
            for f in range(OUT // 16):
                acc = acc + (ehb[j, pl.ds(f * 16, 16)]
                             * etb[j, pl.ds(f * 16, 16)]
                             * wrb[j, pl.ds(f * 16, 16)])
            pb[q * 64 + j] = acc
            return carry2
        lax.fori_loop(0, 64, _triple, 0)
        return carry
    lax.fori_loop(0, 8, _chunk, 0)

    iota16 = lax.iota(_i32, 16)

    def _red(jg, carry):
        rows = iota16 + jg * 16
        tot = jnp.zeros((16,), _f32)
        for f in range(16):
            tot = tot + plsc.load_gather(pb, [rows, jnp.full((16,), f, _i32)])
        sb[pl.ds(jg * 16, 16)] = tot
        return carry
    lax.fori_loop(0, 32, _red, 0)

    pltpu.sync_copy(sb, out_hbm.at[pl.ds(off, 512)])


@functools.cache
def _make_score_kernel():
    mesh = plsc.VectorSubcoreMesh(core_axis_name="c", subcore_axis_name="s")
    return pl.kernel(
        _score_body,
        out_type=jax.ShapeDtypeStruct((B,), _f32),
        mesh=mesh,
        scratch_types=[
            pltpu.VMEM((512,), _i32),
            pltpu.VMEM((512,), _i32),
            pltpu.VMEM((64, OUT), _f32),
            pltpu.VMEM((64, OUT), _f32),
            pltpu.VMEM((64, OUT), _f32),
            pltpu.VMEM((512, 16), _f32),
            pltpu.VMEM((512,), _f32),
            pltpu.SemaphoreType.DMA,
            pltpu.SemaphoreType.DMA,
        ],
        compiler_params=pltpu.CompilerParams(use_tc_tiling_on_sc=False, needs_layout_passes=False),
    )


def _wsel_body(rc_ref, emb_ref, o_ref):
    rc = rc_ref[...]
    out = jnp.zeros((rc.shape[0], OUT), _f32)
    for r in range(NREL):
        out = out + jnp.where(rc == r, 1.0, 0.0) * emb_ref[r][None, :]
    o_ref[...] = out


def _run_wsel(rcol2d, rel_emb):
    TB = 2048
    return pl.pallas_call(
        _wsel_body,
        grid=(B // TB,),
        in_specs=[
            pl.BlockSpec((TB, 1), lambda i: (i, 0)),
            pl.BlockSpec((NREL, OUT), lambda i: (0, 0)),
        ],
        out_specs=pl.BlockSpec((TB, OUT), lambda i: (i, 0)),
        out_shape=jax.ShapeDtypeStruct((B, OUT), _f32),
    )(rcol2d, rel_emb)


def _k1_body(x_ref, w_ref, as_ref, ad_ref, h0_ref, h1_ref, a_ref):
    xb = x_ref[...]
    h = jnp.dot(xb, w_ref[0], preferred_element_type=_f32)
    h0 = h[:, :HID]
    h1 = h[:, HID:]
    h0_ref[0] = h0
    h1_ref[0] = h1
    a_s = as_ref[0, 0]
    a_d = ad_ref[0, 0]
    as0 = (h0 * a_s[0][None, :]).sum(-1)
    as1 = (h1 * a_s[1][None, :]).sum(-1)
    ad0 = (h0 * a_d[0][None, :]).sum(-1)
    ad1 = (h1 * a_d[1][None, :]).sum(-1)
    a_ref[0] = jnp.stack(
        [jnp.stack([as0, ad0], axis=-1), jnp.stack([as1, ad1], axis=-1)],
        axis=1)


def _run_k1(x0, W1, a_src1, a_dst1):
    return pl.pallas_call(
        _k1_body,
        grid=(NREL, N // TN),
        in_specs=[
            pl.BlockSpec((TN, EMB), lambda r, i: (i, 0)),
            pl.BlockSpec((1, EMB, HEADS * HID), lambda r, i: (r, 0, 0)),
            pl.BlockSpec((1, 1, HEADS, HID), lambda r, i: (r, 0, 0, 0)),
            pl.BlockSpec((1, 1, HEADS, HID), lambda r, i: (r, 0, 0, 0)),
        ],
        out_specs=[
            pl.BlockSpec((1, TN, HID), lambda r, i: (r, i, 0)),
            pl.BlockSpec((1, TN, HID), lambda r, i: (r, i, 0)),
            pl.BlockSpec((1, TN, 2, 2), lambda r, i: (r, i, 0, 0)),
        ],
        out_shape=[
            jax.ShapeDtypeStruct((NREL, N, HID), _f32),
            jax.ShapeDtypeStruct((NREL, N, HID), _f32),
            jax.ShapeDtypeStruct((NREL, N, 2, 2), _f32),
        ],
    )(x0, W1, a_src1, a_dst1)


def _post1_body(acc_ref, h0_ref, h1_ref, a_ref, den_ref, b_ref, x_ref):
    at = a_ref[...]
    dt = den_ref[...]
    out = jnp.zeros((at.shape[0], HEADS * HID), _f32)
    for r in range(NREL):
        halves = []
        for c in range(2):
            col = (r * 2 + c) * 2
            asr = at[:, col]
            ads = at[:, col + 1]
            al = asr + ads
            el = jnp.exp(jnp.where(al >= 0.0, al, al * 0.2))
            den = dt[:, r * 2 + c] + el + 1e-16
            hrc = h0_ref[r] if c == 0 else h1_ref[r]
            num = acc_ref[r, c] + hrc * el[:, None]
            halves.append(num / den[:, None])
        row = jnp.concatenate(halves, axis=-1) + b_ref[r][None, :]
        out = out + jnp.where(row > 0.0, row, jnp.exp(row) - 1.0)
    x_ref[...] = out


def _run_post1(acc, h0, h1, a_t, den_t, b1):
    return pl.pallas_call(
        _post1_body,
        grid=(N // TN,),
        in_specs=[
            pl.BlockSpec((NREL, 2, TN, HID), lambda i: (0, 0, i, 0)),
            pl.BlockSpec((NREL, TN, HID), lambda i: (0, i, 0)),
            pl.BlockSpec((NREL, TN, HID), lambda i: (0, i, 0)),
            pl.BlockSpec((TN, 16), lambda i: (i, 0)),
            pl.BlockSpec((TN, 8), lambda i: (i, 0)),
            pl.BlockSpec((NREL, HEADS * HID), lambda i: (0, 0)),
        ],
        out_specs=pl.BlockSpec((TN, HEADS * HID), lambda i: (i, 0)),
        out_shape=jax.ShapeDtypeStruct((N, HEADS * HID), _f32),
    )(acc, h0, h1, a_t, den_t, b1)


def _k2_body(x_ref, w_ref, as_ref, ad_ref, h0_ref, h1_ref, a_ref):
    xb = x_ref[...]
    h = jnp.dot(xb, w_ref[0], preferred_element_type=_f32)
    h0_ref[0] = h[:, :OUT // 2]
    h1_ref[0] = h[:, OUT // 2:]
    asr = (h * as_ref[0, 0, 0][None, :]).sum(-1)
    ads = (h * ad_ref[0, 0, 0][None, :]).sum(-1)
    a_ref[0] = jnp.stack([asr, ads], axis=-1)


def _run_k2(x, W2, a_src2, a_dst2):
    return pl.pallas_call(
        _k2_body,
        grid=(NREL, N // TN),
        in_specs=[
            pl.BlockSpec((TN, HEADS * HID), lambda r, i: (i, 0)),
            pl.BlockSpec((1, HEADS * HID, OUT), lambda r, i: (r, 0, 0)),
            pl.BlockSpec((1, 1, 1, OUT), lambda r, i: (r, 0, 0, 0)),
            pl.BlockSpec((1, 1, 1, OUT), lambda r, i: (r, 0, 0, 0)),
        ],
        out_specs=[
            pl.BlockSpec((1, TN, OUT // 2), lambda r, i: (r, i, 0)),
            pl.BlockSpec((1, TN, OUT // 2), lambda r, i: (r, i, 0)),
            pl.BlockSpec((1, TN, 2), lambda r, i: (r, i, 0)),
        ],
        out_shape=[
            jax.ShapeDtypeStruct((NREL, N, OUT // 2), _f32),
            jax.ShapeDtypeStruct((NREL, N, OUT // 2), _f32),
            jax.ShapeDtypeStruct((NREL, N, 2), _f32),
        ],
    )(x, W2, a_src2, a_dst2)


def _post2_body(acc_ref, h0_ref, h1_ref, a_ref, den_ref, b_ref, x0_ref,
                rw_ref, rb_ref, g_ref, bb_ref, ent_ref):
    at = a_ref[...]
    dt = den_ref[...]
    x2 = jnp.zeros((at.shape[0], OUT), _f32)
    for r in range(NREL):
        asr = at[:, 2 * r]
        ads = at[:, 2 * r + 1]
        al = asr + ads
        el = jnp.exp(jnp.where(al >= 0.0, al, al * 0.2))
        den = dt[:, r] + el + 1e-16
        num = jnp.concatenate(
            [acc_ref[r, 0] + h0_ref[r] * el[:, None],
             acc_ref[r, 1] + h1_ref[r] * el[:, None]], axis=-1)
        x2 = x2 + num / den[:, None] + b_ref[r][None, :]
    pre = x2 + jnp.dot(x0_ref[...], rw_ref[...],
                       preferred_element_type=_f32) + rb_ref[...][None, :]
    mu = pre.mean(axis=-1, keepdims=True)
    d = pre - mu
    var = (d * d).mean(axis=-1, keepdims=True)
    ent_ref[...] = d * lax.rsqrt(var + 1e-5) * g_ref[...][None, :] \
        + bb_ref[...][None, :]


def _run_post2(acc, h0, h1, a_t, den_t, b2, x0, res_W, res_b, ln_g, ln_b):
    return pl.pallas_call(
        _post2_body,
        grid=(N // TN,),
        in_specs=[
            pl.BlockSpec((NREL, 2, TN, OUT // 2), lambda i: (0, 0, i, 0)),
            pl.BlockSpec((NREL, TN, OUT // 2), lambda i: (0, i, 0)),
            pl.BlockSpec((NREL, TN, OUT // 2), lambda i: (0, i, 0)),
            pl.BlockSpec((TN, 8), lambda i: (i, 0)),
            pl.BlockSpec((TN, 4), lambda i: (i, 0)),
            pl.BlockSpec((NREL, OUT), lambda i: (0, 0)),
            pl.BlockSpec((TN, EMB), lambda i: (i, 0)),
            pl.BlockSpec((EMB, OUT), lambda i: (0, 0)),
            pl.BlockSpec((OUT,), lambda i: (0,)),
            pl.BlockSpec((OUT,), lambda i: (0,)),
            pl.BlockSpec((OUT,), lambda i: (0,)),
        ],
        out_specs=pl.BlockSpec((TN, OUT), lambda i: (i, 0)),
        out_shape=jax.ShapeDtypeStruct((N, OUT), _f32),
    )(acc, h0, h1, a_t, den_t, b2, x0, res_W, res_b, ln_g, ln_b)


def kernel(triples, edge_index, entity_emb, W1, a_src1, a_dst1, b1, W2,
           a_src2, a_dst2, b2, res_W, res_b, ln_g, ln_b, rel_emb):
    x0 = entity_emb
    edge_index = edge_index.astype(_i32)

    h1c0, h1c1, aout1 = _run_k1(x0, W1, a_src1, a_dst1)

    edge_fn1 = _make_edge_kernel(HID)
    accs1, dens1 = [], []
    for r in range(NREL):
        acc, den = edge_fn1(edge_index[r, 0], edge_index[r, 1],
                            h1c0[r], h1c1[r],
                            aout1[r, :, 0, 0], aout1[r, :, 1, 0],
                            aout1[r, :, 0, 1], aout1[r, :, 1, 1])
        accs1.append(acc)
        dens1.append(den)
    acc1 = jnp.stack(accs1).reshape(NREL, 2, N, HID)
    a1t = aout1.transpose(1, 0, 2, 3).reshape(N, 16)
    den1t = jnp.stack(dens1).transpose(2, 0, 1).reshape(N, 8)

    x = _run_post1(acc1, h1c0, h1c1, a1t, den1t, b1)

    h2c0, h2c1, aout2 = _run_k2(x, W2, a_src2, a_dst2)

    edge_fn2 = _make_edge_kernel(OUT // 2)
    accs2, dens2 = [], []
    for r in range(NREL):
        acc, den = edge_fn2(edge_index[r, 0], edge_index[r, 1],
                            h2c0[r], h2c1[r],
                            aout2[r, :, 0], aout2[r, :, 0],
                            aout2[r, :, 1], aout2[r, :, 1])
        accs2.append(acc)
        dens2.append(den)
    acc2 = jnp.stack(accs2).reshape(NREL, 2, N, OUT // 2)
    a2t = aout2.transpose(1, 0, 2).reshape(N, 8)
    den2t = jnp.stack(dens2)[:, 0].transpose(1, 0)

    ent = _run_post2(acc2, h2c0, h2c1, a2t, den2t, b2, x0, res_W, res_b,
                     ln_g, ln_b)

    wrow = _run_wsel(triples[:, 1:2].astype(_i32), rel_emb)
    score_fn = _make_score_kernel()
    score = score_fn(triples[:, 0].astype(_i32), triples[:, 2].astype(_i32),
                     wrow, ent)
    return score


# speedup vs baseline: 44.0871x; 1.1832x over previous
import functools

import jax
import jax.numpy as jnp
from jax import lax
from jax.experimental import pallas as pl
from jax.experimental.pallas import tpu as pltpu
from jax.experimental.pallas import tpu_sc as plsc

N = 10000
NREL = 4
E = 80000
EMB = 128
HID = 64
HEADS = 2
OUT = 256
B = 16384

NCHUNK = E // 128
MAXCH = (NCHUNK + 15) // 16
TN = 1000
DEN_PAD = 10240

_i32 = jnp.int32
_f32 = jnp.float32


NCH = 39


def _edge_body(Fh, eis, eid, eisL, eidL,
               h0_0, h0_1, h0_2, h0_3, h1_0, h1_1, h1_2, h1_3,
               as0_0, as0_1, as0_2, as0_3, as1_0, as1_1, as1_2, as1_3,
               ad0_0, ad0_1, ad0_2, ad0_3, ad1_0, ad1_1, ad1_2, ad1_3,
               acc_hbm, den_hbm,
               srcall, dstall, srcbL, dstbL,
               asbA, adbA, echA, gbufA, asbB, adbB, echB, gbufB,
               zbuf, den_sh, acc_sh,
               gsemA, gsemB, asemA, asemB):
    c = lax.axis_index("c")
    s = lax.axis_index("s")
    base = c * N
    zero16 = jnp.zeros((16,), _f32)
    h0s = [h0_0, h0_1, h0_2, h0_3]
    h1s = [h1_0, h1_1, h1_2, h1_3]
    as0s = [as0_0, as0_1, as0_2, as0_3]
    as1s = [as1_0, as1_1, as1_2, as1_3]
    ad0s = [ad0_0, ad0_1, ad0_2, ad0_3]
    ad1s = [ad1_0, ad1_1, ad1_2, ad1_3]
    r0 = s * 625

    def _zero_slab():
        def _zg(i, carry):
            for f in range(Fh // 16):
                gbufA[i, pl.ds(f * 16, 16)] = zero16
            return carry
        lax.fori_loop(0, 128, _zg, 0)

    def _zero_acc():
        for kk in range(4):
            pltpu.sync_copy(gbufA, acc_sh.at[pl.ds(r0 + kk * 128, 128)])
        pltpu.sync_copy(gbufA.at[pl.ds(0, 113)],
                        acc_sh.at[pl.ds(r0 + 512, 113)])
        pltpu.sync_copy(zbuf, den_sh.at[pl.ds(s * 640, 640)])

    def _zz(i, carry):
        zbuf[pl.ds(i * 16, 16)] = zero16
        return carry
    lax.fori_loop(0, 40, _zz, 0)
    _zero_slab()
    _zero_acc()

    for r in range(NREL):
        h0r, h1r = h0s[r], h1s[r]
        as0r, as1r = as0s[r], as1s[r]
        ad0r, ad1r = ad0s[r], ad1s[r]

        pltpu.sync_copy(eis.at[r, :, s], srcall)
        pltpu.sync_copy(eid.at[r, :, s], dstall)

        @pl.when(s == 0)
        def _():
            pltpu.sync_copy(eisL.at[r], srcbL)
            pltpu.sync_copy(eidL.at[r], dstbL)

        plsc.subcore_barrier()

        def _gathers(sref, dref, asb, adb, gbuf, gsem, asem,
                     h0r=h0r, h1r=h1r, as0r=as0r, as1r=as1r,
                     ad0r=ad0r, ad1r=ad1r):
            @pl.when(c == 0)
            def _():
                pltpu.async_copy(h0r.at[sref], gbuf, gsem)
                pltpu.async_copy(as0r.at[sref], asb, asem)
                pltpu.async_copy(ad0r.at[dref], adb, asem)

            @pl.when(c == 1)
            def _():
                pltpu.async_copy(h1r.at[sref], gbuf, gsem)
                pltpu.async_copy(as1r.at[sref], asb, asem)
                pltpu.async_copy(ad1r.at[dref], adb, asem)

        def _process(sref, dref, asb, adb, ech, gbuf, gsem, asem,
                     h0r=h0r, as0r=as0r, ad0r=ad0r):
            pltpu.make_async_copy(as0r.at[sref], asb, asem).wait()
            pltpu.make_async_copy(ad0r.at[dref], adb, asem).wait()
            for g in range(8):
                av = asb[pl.ds(g * 16, 16)] + adb[pl.ds(g * 16, 16)]
                av = jnp.where(av >= 0.0, av, av * jnp.float32(0.2))
                ech[pl.ds(g * 16, 16)] = jnp.exp(av)

            pltpu.sync_copy(ech, den_sh.at[dref], add=True)
            pltpu.make_async_copy(h0r.at[sref], gbuf, gsem).wait()

            @plsc.parallel_loop(0, 128, step=2, unroll=4)
            def _scale(rw):
                ev = plsc.load_gather(ech, [jnp.full((16,), rw, _i32)])
                ev2 = plsc.load_gather(ech, [jnp.full((16,), rw + 1, _i32)])
                for f in range(Fh // 16):
                    gbuf[rw, pl.ds(f * 16, 16)] = (
                        gbuf[rw, pl.ds(f * 16, 16)] * ev)
                for f in range(Fh // 16):
                    gbuf[rw + 1, pl.ds(f * 16, 16)] = (
                        gbuf[rw + 1, pl.ds(f * 16, 16)] * ev2)

            pltpu.sync_copy(gbuf, acc_sh.at[dref], add=True)

        _gathers(srcall.at[0], dstall.at[0], asbA, adbA, gbufA, gsemA, asemA)

        def _pair(t, carry):
            j = 2 * t
            _gathers(srcall.at[j + 1], dstall.at[j + 1], asbB, adbB,
                     gbufB, gsemB, asemB)
            _process(srcall.at[j], dstall.at[j], asbA, adbA, echA,
                     gbufA, gsemA, asemA)

            @pl.when(j + 2 < NCH)
            def _():
                _gathers(srcall.at[j + 2], dstall.at[j + 2], asbA, adbA,
                         gbufA, gsemA, asemA)
            _process(srcall.at[j + 1], dstall.at[j + 1], asbB, adbB, echB,
                     gbufB, gsemB, asemB)
            return carry
        lax.fori_loop(0, NCH // 2, _pair, 0)
        _process(srcall.at[NCH - 1], dstall.at[NCH - 1], asbA, adbA, echA,
                 gbufA, gsemA, asemA)

        @pl.when(s == 0)
        def _():
            _gathers(srcbL, dstbL, asbB, adbB, gbufB, gsemB, asemB)
            _process(srcbL, dstbL, asbB, adbB, echB, gbufB, gsemB, asemB)

        plsc.subcore_barrier()

        pltpu.sync_copy(acc_sh.at[pl.ds(r0, 625)],
                        acc_hbm.at[r, pl.ds(base + r0, 625)])

        @pl.when(s == 0)
        def _():
            pltpu.sync_copy(den_sh.at[pl.ds(0, N)], den_hbm.at[r, c])

        if r < NREL - 1:
            _zero_slab()
            _zero_acc()


@functools.cache
def _make_edge_kernel(Fh):
    mesh = plsc.VectorSubcoreMesh(core_axis_name="c", subcore_axis_name="s")
    return pl.kernel(
        functools.partial(_edge_body, Fh),
        out_type=[
            jax.ShapeDtypeStruct((NREL, 2 * N, Fh), _f32),
            jax.ShapeDtypeStruct((NREL, 2, N), _f32),
        ],
        mesh=mesh,
        scratch_types=[
            pltpu.VMEM((NCH, 128), _i32),
            pltpu.VMEM((NCH, 128), _i32),
            pltpu.VMEM((128,), _i32),
            pltpu.VMEM((128,), _i32),
            pltpu.VMEM((128,), _f32),
            pltpu.VMEM((128,), _f32),
            pltpu.VMEM((128,), _f32),
            pltpu.VMEM((128, Fh), _f32),
            pltpu.VMEM((128,), _f32),
            pltpu.VMEM((128,), _f32),
            pltpu.VMEM((128,), _f32),
            pltpu.VMEM((128, Fh), _f32),
            pltpu.VMEM((640,), _f32),
            pltpu.VMEM_SHARED((DEN_PAD,), _f32),
            pltpu.VMEM_SHARED((N, Fh), _f32),
            pltpu.SemaphoreType.DMA,
            pltpu.SemaphoreType.DMA,
            pltpu.SemaphoreType.DMA,
            pltpu.SemaphoreType.DMA,
        ],
        compiler_params=pltpu.CompilerParams(use_tc_tiling_on_sc=False, needs_layout_passes=False),
    )


def _score_body(hcol_hbm, tcol_hbm, wrow_hbm, ent_hbm, out_hbm,
                hl, tl, ehb, etb, wrb, pb, sb, sem1, sem2):
    c = lax.axis_index("c")
    s = lax.axis_index("s")
    w = c * 16 + s
    off = w * 512

    pltpu.sync_copy(hcol_hbm.at[pl.ds(off, 512)], hl)
    pltpu.sync_copy(tcol_hbm.at[pl.ds(off, 512)], tl)

    def _chunk(q, carry):
        cp1 = pltpu.async_copy(ent_hbm.at[hl.at[pl.ds(q * 64, 64)]], ehb, sem1)
        cp2 = pltpu.async_copy(ent_hbm.at[tl.at[pl.ds(q * 64, 64)]], etb, sem2)
        pltpu.sync_copy(wrow_hbm.at[pl.ds(off + q * 64, 64)], wrb)
        cp1.wait()
        cp2.wait()

        def _triple(j, carry2):
            acc = jnp.zeros((16,), _f32)
            for f in range(OUT // 16):
                acc = acc + (ehb[j, pl.ds(f * 16, 16)]
                             * etb[j, pl.ds(f * 16, 16)]
                             * wrb[j, pl.ds(f * 16, 16)])
            pb[q * 64 + j] = acc
            return carry2
        lax.fori_loop(0, 64, _triple, 0)
        return carry
    lax.fori_loop(0, 8, _chunk, 0)

    iota16 = lax.iota(_i32, 16)

    def _red(jg, carry):
        rows = iota16 + jg * 16
        tot = jnp.zeros((16,), _f32)
        for f in range(16):
            tot = tot + plsc.load_gather(pb, [rows, jnp.full((16,), f, _i32)])
        sb[pl.ds(jg * 16, 16)] = tot
        return carry
    lax.fori_loop(0, 32, _red, 0)

    pltpu.sync_copy(sb, out_hbm.at[pl.ds(off, 512)])


@functools.cache
def _make_score_kernel():
    mesh = plsc.VectorSubcoreMesh(core_axis_name="c", subcore_axis_name="s")
    return pl.kernel(
        _score_body,
        out_type=jax.ShapeDtypeStruct((B,), _f32),
        mesh=mesh,
        scratch_types=[
            pltpu.VMEM((512,), _i32),
            pltpu.VMEM((512,), _i32),
            pltpu.VMEM((64, OUT), _f32),
            pltpu.VMEM((64, OUT), _f32),
            pltpu.VMEM((64, OUT), _f32),
            pltpu.VMEM((512, 16), _f32),
            pltpu.VMEM((512,), _f32),
            pltpu.SemaphoreType.DMA,
            pltpu.SemaphoreType.DMA,
        ],
        compiler_params=pltpu.CompilerParams(use_tc_tiling_on_sc=False, needs_layout_passes=False),
    )


def _wsel_body(rc_ref, emb_ref, o_ref):
    rc = rc_ref[...]
    out = jnp.zeros((rc.shape[0], OUT), _f32)
    for r in range(NREL):
        out = out + jnp.where(rc == r, 1.0, 0.0) * emb_ref[r][None, :]
    o_ref[...] = out


def _run_wsel(rcol2d, rel_emb):
    TB = 2048
    return pl.pallas_call(
        _wsel_body,
        grid=(B // TB,),
        in_specs=[
            pl.BlockSpec((TB, 1), lambda i: (i, 0)),
            pl.BlockSpec((NREL, OUT), lambda i: (0, 0)),
        ],
        out_specs=pl.BlockSpec((TB, OUT), lambda i: (i, 0)),
        out_shape=jax.ShapeDtypeStruct((B, OUT), _f32),
    )(rcol2d, rel_emb)


def _k1_body(x_ref, w_ref, as_ref, ad_ref, h0_ref, h1_ref, a_ref):
    xb = x_ref[...]
    h = jnp.dot(xb, w_ref[0], preferred_element_type=_f32)
    h0 = h[:, :HID]
    h1 = h[:, HID:]
    h0_ref[0] = h0
    h1_ref[0] = h1
    a_s = as_ref[0, 0]
    a_d = ad_ref[0, 0]
    as0 = (h0 * a_s[0][None, :]).sum(-1)
    as1 = (h1 * a_s[1][None, :]).sum(-1)
    ad0 = (h0 * a_d[0][None, :]).sum(-1)
    ad1 = (h1 * a_d[1][None, :]).sum(-1)
    a_ref[0] = jnp.stack(
        [jnp.stack([as0, ad0], axis=-1), jnp.stack([as1, ad1], axis=-1)],
        axis=1)


def _run_k1(x0, W1, a_src1, a_dst1):
    return pl.pallas_call(
        _k1_body,
        grid=(NREL, N // TN),
        in_specs=[
            pl.BlockSpec((TN, EMB), lambda r, i: (i, 0)),
            pl.BlockSpec((1, EMB, HEADS * HID), lambda r, i: (r, 0, 0)),
            pl.BlockSpec((1, 1, HEADS, HID), lambda r, i: (r, 0, 0, 0)),
            pl.BlockSpec((1, 1, HEADS, HID), lambda r, i: (r, 0, 0, 0)),
        ],
        out_specs=[
            pl.BlockSpec((1, TN, HID), lambda r, i: (r, i, 0)),
            pl.BlockSpec((1, TN, HID), lambda r, i: (r, i, 0)),
            pl.BlockSpec((1, TN, 2, 2), lambda r, i: (r, i, 0, 0)),
        ],
        out_shape=[
            jax.ShapeDtypeStruct((NREL, N, HID), _f32),
            jax.ShapeDtypeStruct((NREL, N, HID), _f32),
            jax.ShapeDtypeStruct((NREL, N, 2, 2), _f32),
        ],
    )(x0, W1, a_src1, a_dst1)


def _post1_body(acc_ref, h0_ref, h1_ref, a_ref, den_ref, b_ref, x_ref):
    at = a_ref[...]
    dt = den_ref[...]
    out = jnp.zeros((at.shape[0], HEADS * HID), _f32)
    for r in range(NREL):
        halves = []
        for c in range(2):
            col = (r * 2 + c) * 2
            asr = at[:, col]
            ads = at[:, col + 1]
            al = asr + ads
            el = jnp.exp(jnp.where(al >= 0.0, al, al * 0.2))
            den = dt[:, r * 2 + c] + el + 1e-16
            hrc = h0_ref[r] if c == 0 else h1_ref[r]
            num = acc_ref[r, c] + hrc * el[:, None]
            halves.append(num / den[:, None])
        row = jnp.concatenate(halves, axis=-1) + b_ref[r][None, :]
        out = out + jnp.where(row > 0.0, row, jnp.exp(row) - 1.0)
    x_ref[...] = out


def _run_post1(acc, h0, h1, a_t, den_t, b1):
    return pl.pallas_call(
        _post1_body,
        grid=(N // TN,),
        in_specs=[
            pl.BlockSpec((NREL, 2, TN, HID), lambda i: (0, 0, i, 0)),
            pl.BlockSpec((NREL, TN, HID), lambda i: (0, i, 0)),
            pl.BlockSpec((NREL, TN, HID), lambda i: (0, i, 0)),
            pl.BlockSpec((TN, 16), lambda i: (i, 0)),
            pl.BlockSpec((TN, 8), lambda i: (i, 0)),
            pl.BlockSpec((NREL, HEADS * HID), lambda i: (0, 0)),
        ],
        out_specs=pl.BlockSpec((TN, HEADS * HID), lambda i: (i, 0)),
        out_shape=jax.ShapeDtypeStruct((N, HEADS * HID), _f32),
    )(acc, h0, h1, a_t, den_t, b1)


def _k2_body(x_ref, w_ref, as_ref, ad_ref, h0_ref, h1_ref, a_ref):
    xb = x_ref[...]
    h = jnp.dot(xb, w_ref[0], preferred_element_type=_f32)
    h0_ref[0] = h[:, :OUT // 2]
    h1_ref[0] = h[:, OUT // 2:]
    asr = (h * as_ref[0, 0, 0][None, :]).sum(-1)
    ads = (h * ad_ref[0, 0, 0][None, :]).sum(-1)
    a_ref[0] = jnp.stack([asr, ads], axis=-1)


def _run_k2(x, W2, a_src2, a_dst2):
    return pl.pallas_call(
        _k2_body,
        grid=(NREL, N // TN),
        in_specs=[
            pl.BlockSpec((TN, HEADS * HID), lambda r, i: (i, 0)),
            pl.BlockSpec((1, HEADS * HID, OUT), lambda r, i: (r, 0, 0)),
            pl.BlockSpec((1, 1, 1, OUT), lambda r, i: (r, 0, 0, 0)),
            pl.BlockSpec((1, 1, 1, OUT), lambda r, i: (r, 0, 0, 0)),
        ],
        out_specs=[
            pl.BlockSpec((1, TN, OUT // 2), lambda r, i: (r, i, 0)),
            pl.BlockSpec((1, TN, OUT // 2), lambda r, i: (r, i, 0)),
            pl.BlockSpec((1, TN, 2), lambda r, i: (r, i, 0)),
        ],
        out_shape=[
            jax.ShapeDtypeStruct((NREL, N, OUT // 2), _f32),
            jax.ShapeDtypeStruct((NREL, N, OUT // 2), _f32),
            jax.ShapeDtypeStruct((NREL, N, 2), _f32),
        ],
    )(x, W2, a_src2, a_dst2)


def _post2_body(acc_ref, h0_ref, h1_ref, a_ref, den_ref, b_ref, x0_ref,
                rw_ref, rb_ref, g_ref, bb_ref, ent_ref):
    at = a_ref[...]
    dt = den_ref[...]
    x2 = jnp.zeros((at.shape[0], OUT), _f32)
    for r in range(NREL):
        asr = at[:, 2 * r]
        ads = at[:, 2 * r + 1]
        al = asr + ads
        el = jnp.exp(jnp.where(al >= 0.0, al, al * 0.2))
        den = dt[:, r] + el + 1e-16
        num = jnp.concatenate(
            [acc_ref[r, 0] + h0_ref[r] * el[:, None],
             acc_ref[r, 1] + h1_ref[r] * el[:, None]], axis=-1)
        x2 = x2 + num / den[:, None] + b_ref[r][None, :]
    pre = x2 + jnp.dot(x0_ref[...], rw_ref[...],
                       preferred_element_type=_f32) + rb_ref[...][None, :]
    mu = pre.mean(axis=-1, keepdims=True)
    d = pre - mu
    var = (d * d).mean(axis=-1, keepdims=True)
    ent_ref[...] = d * lax.rsqrt(var + 1e-5) * g_ref[...][None, :] \
        + bb_ref[...][None, :]


def _run_post2(acc, h0, h1, a_t, den_t, b2, x0, res_W, res_b, ln_g, ln_b):
    return pl.pallas_call(
        _post2_body,
        grid=(N // TN,),
        in_specs=[
            pl.BlockSpec((NREL, 2, TN, OUT // 2), lambda i: (0, 0, i, 0)),
            pl.BlockSpec((NREL, TN, OUT // 2), lambda i: (0, i, 0)),
            pl.BlockSpec((NREL, TN, OUT // 2), lambda i: (0, i, 0)),
            pl.BlockSpec((TN, 8), lambda i: (i, 0)),
            pl.BlockSpec((TN, 4), lambda i: (i, 0)),
            pl.BlockSpec((NREL, OUT), lambda i: (0, 0)),
            pl.BlockSpec((TN, EMB), lambda i: (i, 0)),
            pl.BlockSpec((EMB, OUT), lambda i: (0, 0)),
            pl.BlockSpec((OUT,), lambda i: (0,)),
            pl.BlockSpec((OUT,), lambda i: (0,)),
            pl.BlockSpec((OUT,), lambda i: (0,)),
        ],
        out_specs=pl.BlockSpec((TN, OUT), lambda i: (i, 0)),
        out_shape=jax.ShapeDtypeStruct((N, OUT), _f32),
    )(acc, h0, h1, a_t, den_t, b2, x0, res_W, res_b, ln_g, ln_b)


def kernel(triples, edge_index, entity_emb, W1, a_src1, a_dst1, b1, W2,
           a_src2, a_dst2, b2, res_W, res_b, ln_g, ln_b, rel_emb):
    x0 = entity_emb
    edge_index = edge_index.astype(_i32)

    h1c0, h1c1, aout1 = _run_k1(x0, W1, a_src1, a_dst1)

    eis = edge_index[:, 0, :624 * 128].reshape(NREL, NCH, 16, 128)
    eid = edge_index[:, 1, :624 * 128].reshape(NREL, NCH, 16, 128)
    eisL = edge_index[:, 0, 624 * 128:]
    eidL = edge_index[:, 1, 624 * 128:]

    edge_fn1 = _make_edge_kernel(HID)
    acc1s, den1 = edge_fn1(
        eis, eid, eisL, eidL,
        h1c0[0], h1c0[1], h1c0[2], h1c0[3],
        h1c1[0], h1c1[1], h1c1[2], h1c1[3],
        aout1[0, :, 0, 0], aout1[1, :, 0, 0], aout1[2, :, 0, 0], aout1[3, :, 0, 0],
        aout1[0, :, 1, 0], aout1[1, :, 1, 0], aout1[2, :, 1, 0], aout1[3, :, 1, 0],
        aout1[0, :, 0, 1], aout1[1, :, 0, 1], aout1[2, :, 0, 1], aout1[3, :, 0, 1],
        aout1[0, :, 1, 1], aout1[1, :, 1, 1], aout1[2, :, 1, 1], aout1[3, :, 1, 1])
    acc1 = acc1s.reshape(NREL, 2, N, HID)
    a1t = aout1.transpose(1, 0, 2, 3).reshape(N, 16)
    den1t = den1.transpose(2, 0, 1).reshape(N, 8)

    x = _run_post1(acc1, h1c0, h1c1, a1t, den1t, b1)

    h2c0, h2c1, aout2 = _run_k2(x, W2, a_src2, a_dst2)

    edge_fn2 = _make_edge_kernel(OUT // 2)
    acc2s, den2 = edge_fn2(
        eis, eid, eisL, eidL,
        h2c0[0], h2c0[1], h2c0[2], h2c0[3],
        h2c1[0], h2c1[1], h2c1[2], h2c1[3],
        aout2[0, :, 0], aout2[1, :, 0], aout2[2, :, 0], aout2[3, :, 0],
        aout2[0, :, 0], aout2[1, :, 0], aout2[2, :, 0], aout2[3, :, 0],
        aout2[0, :, 1], aout2[1, :, 1], aout2[2, :, 1], aout2[3, :, 1],
        aout2[0, :, 1], aout2[1, :, 1], aout2[2, :, 1], aout2[3, :, 1])
    acc2 = acc2s.reshape(NREL, 2, N, OUT // 2)
    a2t = aout2.transpose(1, 0, 2).reshape(N, 8)
    den2t = den2[:, 0].transpose(1, 0)

    ent = _run_post2(acc2, h2c0, h2c1, a2t, den2t, b2, x0, res_W, res_b,
                     ln_g, ln_b)

    wrow = _run_wsel(triples[:, 1:2].astype(_i32), rel_emb)
    score_fn = _make_score_kernel()
    score = score_fn(triples[:, 0].astype(_i32), triples[:, 2].astype(_i32),
                     wrow, ent)
    return score
